# baseline (speedup 1.0000x reference)
import functools

import jax
import jax.numpy as jnp
from jax import lax
from jax.experimental import pallas as pl
from jax.experimental.pallas import tpu as pltpu
from jax.experimental.pallas import tpu_sc as plsc

N_NODES = 10000
N_EDGES = 320000
D_FEAT = 128
S_SUB = 50000
N_GRAPHS = 256
EMB = 64
HID = 64

N_TILES = 32

NP = 10240
NROWS_PER_TILE = NP // 16
DUMMY_NODE = N_NODES

EDGE_CHUNK = 128
EDGE_GRP = 4
EDGE_PAIRS = 10
EDGE_CHUNKS = EDGE_PAIRS * 2 * EDGE_GRP
EDGES_PER_TILE = EDGE_CHUNKS * EDGE_CHUNK
EP = EDGES_PER_TILE * N_TILES

SUB_CHUNK = 64
SUB_GRP = 4
SUB_PAIRS = 3
SUB_CHUNKS = SUB_PAIRS * 2 * SUB_GRP + 1
SUB_PER_TILE = SUB_CHUNKS * SUB_CHUNK
SP = SUB_PER_TILE * N_TILES

BIN_REP = 4
GP = 1152
GROWS_PER_TILE = GP // 16

ROW_BLK = 2048


def _sc_mesh():
    return plsc.VectorSubcoreMesh(core_axis_name="c", subcore_axis_name="s")


_SC_PARAMS = pltpu.CompilerParams(use_tc_tiling_on_sc=False)


@functools.partial(
    pl.kernel,
    out_type=jax.ShapeDtypeStruct((2, NP, 8), jnp.float32),
    mesh=_sc_mesh(),
    compiler_params=_SC_PARAMS,
    scratch_types=[
        pltpu.VMEM((EDGE_CHUNKS, EDGE_CHUNK), jnp.int32),
        pltpu.VMEM((EDGE_CHUNK, 8), jnp.float32),
        pltpu.SemaphoreType.DMA,
        pltpu.VMEM_SHARED((NP, 8), jnp.float32),
    ],
)
def _deg_kernel(dst_hbm, ones_hbm, zeros_hbm, out_hbm, dst_all, ones_v, sem, deg_sh):
    c = lax.axis_index("c")
    s = lax.axis_index("s")
    wid = c * 16 + s
    pltpu.sync_copy(zeros_hbm, deg_sh.at[pl.ds(s * NROWS_PER_TILE, NROWS_PER_TILE)])
    pltpu.sync_copy(dst_hbm.at[pl.ds(wid * EDGE_CHUNKS, EDGE_CHUNKS)], dst_all)
    pltpu.sync_copy(ones_hbm, ones_v)
    plsc.subcore_barrier()

    def body(gp, _):
        base = gp * 2 * EDGE_GRP
        ds = [pltpu.async_copy(ones_v, deg_sh.at[dst_all.at[base + k]], sem,
                               add=True)
              for k in range(2 * EDGE_GRP)]
        for d in ds:
            d.wait()
        return 0

    lax.fori_loop(0, EDGE_PAIRS, body, 0)
    plsc.subcore_barrier()
    r0 = s * NROWS_PER_TILE
    pltpu.sync_copy(deg_sh.at[pl.ds(r0, NROWS_PER_TILE)],
                    out_hbm.at[c, pl.ds(r0, NROWS_PER_TILE)])


def _enc_body(dp_ref, x_ref, w_ref, y_ref, dinv_ref):
    dp = dp_ref[...]
    deg = dp[0] + dp[1] + 1.0
    dinv = lax.rsqrt(deg)
    xw = jnp.dot(x_ref[...], w_ref[...], preferred_element_type=jnp.float32)
    y_ref[...] = xw * dinv[:, 0:1]
    dinv_ref[...] = dinv


def _enc_call(deg_parts, x_pad, w_enc):
    grid = NP // ROW_BLK
    return pl.pallas_call(
        _enc_body,
        grid=(grid, 2),
        in_specs=[
            pl.BlockSpec((2, ROW_BLK, 8), lambda i, j: (0, i, 0)),
            pl.BlockSpec((ROW_BLK, D_FEAT), lambda i, j: (i, 0)),
            pl.BlockSpec((D_FEAT, EMB), lambda i, j: (0, 0)),
        ],
        out_specs=[
            pl.BlockSpec((ROW_BLK, EMB), lambda i, j: (j * grid + i, 0)),
            pl.BlockSpec((ROW_BLK, 8), lambda i, j: (i, 0)),
        ],
        out_shape=[
            jax.ShapeDtypeStruct((2 * NP, EMB), jnp.float32),
            jax.ShapeDtypeStruct((NP, 8), jnp.float32),
        ],
    )(deg_parts, x_pad, w_enc)


@functools.partial(
    pl.kernel,
    out_type=jax.ShapeDtypeStruct((2, NP, EMB), jnp.float32),
    mesh=_sc_mesh(),
    compiler_params=_SC_PARAMS,
    scratch_types=[
        pltpu.VMEM((EDGES_PER_TILE,), jnp.int32),
        pltpu.VMEM((EDGE_CHUNKS, EDGE_CHUNK), jnp.int32),
        [pltpu.VMEM((EDGE_CHUNK, EMB), jnp.float32) for _ in range(EDGE_GRP)],
        [pltpu.VMEM((EDGE_CHUNK, EMB), jnp.float32) for _ in range(EDGE_GRP)],
        pltpu.SemaphoreType.DMA,
        pltpu.SemaphoreType.DMA,
        pltpu.SemaphoreType.DMA,
        pltpu.SemaphoreType.DMA,
        pltpu.VMEM_SHARED((NP, EMB), jnp.float32),
    ],
)
def _edge_kernel(src_hbm, dst_hbm, y_hbm, zeros_hbm, out_hbm,
                 src_all, dst_all, bufs_a, bufs_b, sga, sgb, ssa, ssb, acc_sh):
    c = lax.axis_index("c")
    s = lax.axis_index("s")
    wid = c * 16 + s
    pltpu.sync_copy(zeros_hbm, acc_sh.at[pl.ds(s * NROWS_PER_TILE, NROWS_PER_TILE)])
    pltpu.sync_copy(src_hbm.at[wid], src_all)
    pltpu.sync_copy(dst_hbm.at[pl.ds(wid * EDGE_CHUNKS, EDGE_CHUNKS)], dst_all)

    def fire_gather(chunk, buf, sem):
        idx = src_all.at[pl.ds(chunk * EDGE_CHUNK, EDGE_CHUNK)]
        return pltpu.async_copy(y_hbm.at[idx], buf, sem)

    def fire_scatter(chunk, buf, sem):
        return pltpu.async_copy(buf, acc_sh.at[dst_all.at[chunk]], sem, add=True)

    plsc.subcore_barrier()

    def body(gp, _):
        base_a = gp * 2 * EDGE_GRP
        base_b = base_a + EDGE_GRP
        ga = [fire_gather(base_a + k, bufs_a[k], sga) for k in range(EDGE_GRP)]
        gb = [fire_gather(base_b + k, bufs_b[k], sgb) for k in range(EDGE_GRP)]
        for d in ga:
            d.wait()
        sa = [fire_scatter(base_a + k, bufs_a[k], ssa) for k in range(EDGE_GRP)]
        for d in gb:
            d.wait()
        sb = [fire_scatter(base_b + k, bufs_b[k], ssb) for k in range(EDGE_GRP)]
        for d in sa:
            d.wait()
        for d in sb:
            d.wait()
        return 0

    lax.fori_loop(0, EDGE_PAIRS, body, 0)
    plsc.subcore_barrier()
    r0 = s * NROWS_PER_TILE
    pltpu.sync_copy(acc_sh.at[pl.ds(r0, NROWS_PER_TILE)],
                    out_hbm.at[c, pl.ds(r0, NROWS_PER_TILE)])


def _combine_body(a_ref, y_ref, dinv_ref, b_ref, h_ref):
    a = a_ref[...]
    acc = a[0] + a[1] + y_ref[...]
    h_ref[...] = jnp.maximum(acc * dinv_ref[...][:, 0:1] + b_ref[...], 0.0)


def _combine_call(acc_parts, y2, dinv, b_enc2d):
    grid = NP // ROW_BLK
    return pl.pallas_call(
        _combine_body,
        grid=(grid, 2),
        in_specs=[
            pl.BlockSpec((2, ROW_BLK, EMB), lambda i, j: (0, i, 0)),
            pl.BlockSpec((ROW_BLK, EMB), lambda i, j: (i, 0)),
            pl.BlockSpec((ROW_BLK, 8), lambda i, j: (i, 0)),
            pl.BlockSpec((1, EMB), lambda i, j: (0, 0)),
        ],
        out_specs=pl.BlockSpec((ROW_BLK, EMB), lambda i, j: (j * grid + i, 0)),
        out_shape=jax.ShapeDtypeStruct((2 * NP, EMB), jnp.float32),
    )(acc_parts, y2, dinv, b_enc2d)


@functools.partial(
    pl.kernel,
    out_type=[
        jax.ShapeDtypeStruct((2, GP, EMB), jnp.float32),
        jax.ShapeDtypeStruct((2, GP, 8), jnp.float32),
    ],
    mesh=_sc_mesh(),
    compiler_params=_SC_PARAMS,
    scratch_types=[
        pltpu.VMEM((SUB_PER_TILE,), jnp.int32),
        pltpu.VMEM((SUB_CHUNKS, SUB_CHUNK), jnp.int32),
        [pltpu.VMEM((SUB_CHUNK, EMB), jnp.float32) for _ in range(SUB_GRP)],
        [pltpu.VMEM((SUB_CHUNK, EMB), jnp.float32) for _ in range(SUB_GRP)],
        pltpu.VMEM((SUB_CHUNK, 8), jnp.float32),
        pltpu.SemaphoreType.DMA,
        pltpu.SemaphoreType.DMA,
        pltpu.SemaphoreType.DMA,
        pltpu.SemaphoreType.DMA,
        pltpu.VMEM_SHARED((GP, EMB), jnp.float32),
        pltpu.VMEM_SHARED((GP, 8), jnp.float32),
    ],
)
def _pool_kernel(sub_hbm, batch_hbm, h_hbm, zp_hbm, zc_hbm, ones_hbm,
                 pool_hbm, cnt_hbm,
                 sub_all, bat_all, bufs_a, bufs_b, ones_v,
                 sga, sgb, ssa, ssb, pool_sh, cnt_sh):
    c = lax.axis_index("c")
    s = lax.axis_index("s")
    wid = c * 16 + s
    g0 = s * GROWS_PER_TILE
    pltpu.sync_copy(zp_hbm, pool_sh.at[pl.ds(g0, GROWS_PER_TILE)])
    pltpu.sync_copy(zc_hbm, cnt_sh.at[pl.ds(g0, GROWS_PER_TILE)])
    pltpu.sync_copy(sub_hbm.at[wid], sub_all)
    pltpu.sync_copy(batch_hbm.at[pl.ds(wid * SUB_CHUNKS, SUB_CHUNKS)], bat_all)
    pltpu.sync_copy(ones_hbm, ones_v)

    def fire_gather(chunk, buf, sem):
        idx = sub_all.at[pl.ds(chunk * SUB_CHUNK, SUB_CHUNK)]
        return pltpu.async_copy(h_hbm.at[idx], buf, sem)

    def fire_scatter(chunk, buf, sem):
        d1 = pltpu.async_copy(buf, pool_sh.at[bat_all.at[chunk]], sem, add=True)
        d2 = pltpu.async_copy(ones_v, cnt_sh.at[bat_all.at[chunk]], sem, add=True)
        return (d1, d2)

    plsc.subcore_barrier()

    def body(gp, _):
        base_a = gp * 2 * SUB_GRP
        base_b = base_a + SUB_GRP
        ga = [fire_gather(base_a + k, bufs_a[k], sga) for k in range(SUB_GRP)]
        gb = [fire_gather(base_b + k, bufs_b[k], sgb) for k in range(SUB_GRP)]
        for d in ga:
            d.wait()
        sa = [fire_scatter(base_a + k, bufs_a[k], ssa) for k in range(SUB_GRP)]
        for d in gb:
            d.wait()
        sb = [fire_scatter(base_b + k, bufs_b[k], ssb) for k in range(SUB_GRP)]
        for d1, d2 in sa:
            d1.wait()
            d2.wait()
        for d1, d2 in sb:
            d1.wait()
            d2.wait()
        return 0

    lax.fori_loop(0, SUB_PAIRS, body, 0)
    tg = fire_gather(SUB_CHUNKS - 1, bufs_a[0], sga)
    tg.wait()
    t1, t2 = fire_scatter(SUB_CHUNKS - 1, bufs_a[0], ssa)
    t1.wait()
    t2.wait()
    plsc.subcore_barrier()
    pltpu.sync_copy(pool_sh.at[pl.ds(g0, GROWS_PER_TILE)],
                    pool_hbm.at[c, pl.ds(g0, GROWS_PER_TILE)])
    pltpu.sync_copy(cnt_sh.at[pl.ds(g0, GROWS_PER_TILE)],
                    cnt_hbm.at[c, pl.ds(g0, GROWS_PER_TILE)])


def _head_body(pool_ref, cnt_ref, w1_ref, b1_ref, g_ref, be_ref, w2_ref, b2_ref,
               out_ref):
    pp = pool_ref[...]
    cc = cnt_ref[...]
    nreal = N_GRAPHS * BIN_REP
    pool = (pp[0, :nreal, :] + pp[1, :nreal, :]).reshape(
        N_GRAPHS, BIN_REP, EMB).sum(axis=1)
    cnt = (cc[0, :nreal, :] + cc[1, :nreal, :]).reshape(
        N_GRAPHS, BIN_REP, 8).sum(axis=1)[:, 0:1]
    pooled = pool / jnp.maximum(cnt, 1.0)
    z = jnp.dot(pooled, w1_ref[...], preferred_element_type=jnp.float32) + b1_ref[...]
    z = jnp.maximum(z, 0.0)
    mu = jnp.mean(z, axis=0, keepdims=True)
    var = jnp.mean((z - mu) ** 2, axis=0, keepdims=True)
    z = (z - mu) / jnp.sqrt(var + 1e-5) * g_ref[...] + be_ref[...]
    out_ref[...] = jnp.sum(z * w2_ref[...], axis=1, keepdims=True) + b2_ref[...]


def _head_call(pool_parts, cnt_parts, w1, b1_2d, gamma2d, beta2d, w2row, b2_2d):
    return pl.pallas_call(
        _head_body,
        out_shape=jax.ShapeDtypeStruct((N_GRAPHS, 1), jnp.float32),
    )(pool_parts, cnt_parts, w1, b1_2d, gamma2d, beta2d, w2row, b2_2d)


def kernel(full_x, full_edge_index, subgraph_node_indices, batch_vector,
           W_enc, b_enc, W1, b1, gamma, beta, W2, b2):
    src = full_edge_index[0]
    dst = full_edge_index[1]
    e_pad = EP - N_EDGES
    src_pad = jnp.concatenate([src, jnp.zeros((e_pad,), jnp.int32)])
    dst_fill = N_NODES + (jnp.arange(e_pad, dtype=jnp.int32) % (NP - N_NODES))
    dst_pad = jnp.concatenate([dst, dst_fill])
    sc1_shift = jnp.where(jnp.arange(N_TILES, dtype=jnp.int32) >= 16, NP, 0)[:, None]
    src_2d = src_pad.reshape(N_TILES, EDGES_PER_TILE) + sc1_shift
    dst_2d = dst_pad.reshape(N_TILES * EDGE_CHUNKS, EDGE_CHUNK)
    s_pad = SP - S_SUB
    sub_pad = jnp.concatenate([subgraph_node_indices, jnp.zeros((s_pad,), jnp.int32)])
    bins_real = batch_vector * BIN_REP + (jnp.arange(S_SUB, dtype=jnp.int32) % BIN_REP)
    bins_pad = N_GRAPHS * BIN_REP + (jnp.arange(s_pad, dtype=jnp.int32) % (GP - N_GRAPHS * BIN_REP))
    batch_pad = jnp.concatenate([bins_real, bins_pad])
    sub_2d = sub_pad.reshape(N_TILES, SUB_PER_TILE) + sc1_shift
    batch_2d = batch_pad.reshape(N_TILES * SUB_CHUNKS, SUB_CHUNK)
    x_pad = jnp.pad(full_x, ((0, NP - N_NODES), (0, 0)))

    ones_e = jnp.ones((EDGE_CHUNK, 8), jnp.float32)
    zeros_deg = jnp.zeros((NROWS_PER_TILE, 8), jnp.float32)
    zeros_acc = jnp.zeros((NROWS_PER_TILE, EMB), jnp.float32)
    ones_s = jnp.ones((SUB_CHUNK, 8), jnp.float32)
    zeros_pool = jnp.zeros((GROWS_PER_TILE, EMB), jnp.float32)
    zeros_cnt = jnp.zeros((GROWS_PER_TILE, 8), jnp.float32)

    deg_parts = _deg_kernel(dst_2d, ones_e, zeros_deg)
    y, dinv = _enc_call(deg_parts, x_pad, W_enc)
    acc_parts = _edge_kernel(src_2d, dst_2d, y, zeros_acc)
    h = _combine_call(acc_parts, y, dinv, b_enc.reshape(1, EMB))
    pool_parts, cnt_parts = _pool_kernel(sub_2d, batch_2d, h,
                                         zeros_pool, zeros_cnt, ones_s)
    return _head_call(pool_parts, cnt_parts, W1, b1.reshape(1, HID),
                      gamma.reshape(1, HID), beta.reshape(1, HID),
                      W2.reshape(1, HID), b2.reshape(1, 1))


# speedup vs baseline: 2.2281x; 2.2281x over previous
import functools

import jax
import jax.numpy as jnp
from jax import lax
from jax.experimental import pallas as pl
from jax.experimental.pallas import tpu as pltpu
from jax.experimental.pallas import tpu_sc as plsc

N_NODES = 10000
N_EDGES = 320000
D_FEAT = 128
S_SUB = 50000
N_GRAPHS = 256
EMB = 64
HID = 64

N_TILES = 32

NP = 10240
NROWS_PER_TILE = NP // 16
DUMMY_NODE = N_NODES

EDGE_CHUNK = 128
EDGE_GRP = 2
EDGE_PAIRS = 20
EDGE_CHUNKS = EDGE_PAIRS * 2 * EDGE_GRP
EDGES_PER_TILE = EDGE_CHUNKS * EDGE_CHUNK
EP = EDGES_PER_TILE * N_TILES

SUB_CHUNK = 64
SUB_GRP = 4
SUB_PAIRS = 3
SUB_CHUNKS = SUB_PAIRS * 2 * SUB_GRP + 1
SUB_PER_TILE = SUB_CHUNKS * SUB_CHUNK
SP = SUB_PER_TILE * N_TILES

BIN_REP = 4
GP = 1152
GROWS_PER_TILE = GP // 16

ROW_BLK = 2048


def _sc_mesh():
    return plsc.VectorSubcoreMesh(core_axis_name="c", subcore_axis_name="s")


_SC_PARAMS = pltpu.CompilerParams(use_tc_tiling_on_sc=False)


@functools.partial(
    pl.kernel,
    out_type=jax.ShapeDtypeStruct((2, NP, 8), jnp.float32),
    mesh=_sc_mesh(),
    compiler_params=_SC_PARAMS,
    scratch_types=[
        pltpu.VMEM((EDGE_CHUNKS, EDGE_CHUNK), jnp.int32),
        pltpu.VMEM((EDGE_CHUNK, 8), jnp.float32),
        pltpu.SemaphoreType.DMA,
        pltpu.VMEM_SHARED((NP, 8), jnp.float32),
    ],
)
def _deg_kernel(dst_hbm, ones_hbm, zeros_hbm, out_hbm, dst_all, ones_v, sem, deg_sh):
    c = lax.axis_index("c")
    s = lax.axis_index("s")
    wid = c * 16 + s
    pltpu.sync_copy(zeros_hbm, deg_sh.at[pl.ds(s * NROWS_PER_TILE, NROWS_PER_TILE)])
    pltpu.sync_copy(dst_hbm.at[pl.ds(wid * EDGE_CHUNKS, EDGE_CHUNKS)], dst_all)
    pltpu.sync_copy(ones_hbm, ones_v)
    plsc.subcore_barrier()

    def body(gp, _):
        base = gp * 2 * EDGE_GRP
        ds = [pltpu.async_copy(ones_v, deg_sh.at[dst_all.at[base + k]], sem,
                               add=True)
              for k in range(2 * EDGE_GRP)]
        for d in ds:
            d.wait()
        return 0

    lax.fori_loop(0, EDGE_PAIRS, body, 0)
    plsc.subcore_barrier()
    r0 = s * NROWS_PER_TILE
    pltpu.sync_copy(deg_sh.at[pl.ds(r0, NROWS_PER_TILE)],
                    out_hbm.at[c, pl.ds(r0, NROWS_PER_TILE)])


def _enc_body(dp_ref, x_ref, w_ref, y_ref, dinv_ref):
    dp = dp_ref[...]
    deg = dp[0] + dp[1] + 1.0
    dinv = lax.rsqrt(deg)
    xw = jnp.dot(x_ref[...], w_ref[...], preferred_element_type=jnp.float32)
    y_ref[...] = xw * dinv[:, 0:1]
    dinv_ref[...] = dinv


def _enc_call(deg_parts, x_pad, w_enc):
    grid = NP // ROW_BLK
    return pl.pallas_call(
        _enc_body,
        grid=(grid,),
        in_specs=[
            pl.BlockSpec((2, ROW_BLK, 8), lambda i: (0, i, 0)),
            pl.BlockSpec((ROW_BLK, D_FEAT), lambda i: (i, 0)),
            pl.BlockSpec((D_FEAT, EMB), lambda i: (0, 0)),
        ],
        out_specs=[
            pl.BlockSpec((ROW_BLK, EMB), lambda i: (i, 0)),
            pl.BlockSpec((ROW_BLK, 8), lambda i: (i, 0)),
        ],
        out_shape=[
            jax.ShapeDtypeStruct((NP, EMB), jnp.float32),
            jax.ShapeDtypeStruct((NP, 8), jnp.float32),
        ],
    )(deg_parts, x_pad, w_enc)


@functools.partial(
    pl.kernel,
    out_type=jax.ShapeDtypeStruct((2, NP, EMB), jnp.float32),
    mesh=_sc_mesh(),
    compiler_params=_SC_PARAMS,
    scratch_types=[
        pltpu.VMEM((EDGES_PER_TILE,), jnp.int32),
        [pltpu.VMEM((EDGE_CHUNK,), jnp.int32) for _ in range(EDGE_GRP)],
        [pltpu.VMEM((EDGE_CHUNK,), jnp.int32) for _ in range(EDGE_GRP)],
        [pltpu.VMEM((EDGE_CHUNK, EMB), jnp.float32) for _ in range(EDGE_GRP)],
        [pltpu.VMEM((EDGE_CHUNK, EMB), jnp.float32) for _ in range(EDGE_GRP)],
        pltpu.SemaphoreType.DMA,
        pltpu.SemaphoreType.DMA,
        pltpu.SemaphoreType.DMA,
        pltpu.SemaphoreType.DMA,
        pltpu.SemaphoreType.DMA,
        pltpu.SemaphoreType.DMA,
        pltpu.VMEM_SHARED((NP, EMB), jnp.float32),
        pltpu.VMEM_SHARED((NP, EMB), jnp.float32),
    ],
)
def _edge_kernel(src_hbm, dst_hbm, y_hbm, zeros_hbm, out_hbm,
                 src_all, dbuf_a, dbuf_b, bufs_a, bufs_b,
                 sga, sgb, ssa, ssb, sda, sdb, acc_sh, y_sh):
    c = lax.axis_index("c")
    s = lax.axis_index("s")
    wid = c * 16 + s
    r0s = s * NROWS_PER_TILE
    pltpu.sync_copy(zeros_hbm, acc_sh.at[pl.ds(r0s, NROWS_PER_TILE)])
    pltpu.sync_copy(y_hbm.at[pl.ds(r0s, NROWS_PER_TILE)],
                    y_sh.at[pl.ds(r0s, NROWS_PER_TILE)])
    pltpu.sync_copy(src_hbm.at[wid], src_all)

    def fire_gather(chunk, buf, sem):
        idx = src_all.at[pl.ds(chunk * EDGE_CHUNK, EDGE_CHUNK)]
        return pltpu.async_copy(y_sh.at[idx], buf, sem)

    plsc.subcore_barrier()

    def body(gp, _):
        base_a = gp * 2 * EDGE_GRP
        base_b = base_a + EDGE_GRP
        da = [pltpu.async_copy(dst_hbm.at[wid * EDGE_CHUNKS + base_a + k],
                               dbuf_a[k], sda) for k in range(EDGE_GRP)]
        db = [pltpu.async_copy(dst_hbm.at[wid * EDGE_CHUNKS + base_b + k],
                               dbuf_b[k], sdb) for k in range(EDGE_GRP)]
        ga = [fire_gather(base_a + k, bufs_a[k], sga) for k in range(EDGE_GRP)]
        gb = [fire_gather(base_b + k, bufs_b[k], sgb) for k in range(EDGE_GRP)]
        for d in ga:
            d.wait()
        for d in da:
            d.wait()
        sa = [pltpu.async_copy(bufs_a[k], acc_sh.at[dbuf_a[k]], ssa, add=True)
              for k in range(EDGE_GRP)]
        for d in gb:
            d.wait()
        for d in db:
            d.wait()
        sb = [pltpu.async_copy(bufs_b[k], acc_sh.at[dbuf_b[k]], ssb, add=True)
              for k in range(EDGE_GRP)]
        for d in sa:
            d.wait()
        for d in sb:
            d.wait()
        return 0

    lax.fori_loop(0, EDGE_PAIRS, body, 0)
    plsc.subcore_barrier()
    r0 = s * NROWS_PER_TILE
    pltpu.sync_copy(acc_sh.at[pl.ds(r0, NROWS_PER_TILE)],
                    out_hbm.at[c, pl.ds(r0, NROWS_PER_TILE)])


def _combine_body(a_ref, y_ref, dinv_ref, b_ref, h_ref):
    a = a_ref[...]
    acc = a[0] + a[1] + y_ref[...]
    h_ref[...] = jnp.maximum(acc * dinv_ref[...][:, 0:1] + b_ref[...], 0.0)


def _combine_call(acc_parts, y, dinv, b_enc2d):
    grid = NP // ROW_BLK
    return pl.pallas_call(
        _combine_body,
        grid=(grid,),
        in_specs=[
            pl.BlockSpec((2, ROW_BLK, EMB), lambda i: (0, i, 0)),
            pl.BlockSpec((ROW_BLK, EMB), lambda i: (i, 0)),
            pl.BlockSpec((ROW_BLK, 8), lambda i: (i, 0)),
            pl.BlockSpec((1, EMB), lambda i: (0, 0)),
        ],
        out_specs=pl.BlockSpec((ROW_BLK, EMB), lambda i: (i, 0)),
        out_shape=jax.ShapeDtypeStruct((NP, EMB), jnp.float32),
    )(acc_parts, y, dinv, b_enc2d)


@functools.partial(
    pl.kernel,
    out_type=[
        jax.ShapeDtypeStruct((2, GP, EMB), jnp.float32),
        jax.ShapeDtypeStruct((2, GP, 8), jnp.float32),
    ],
    mesh=_sc_mesh(),
    compiler_params=_SC_PARAMS,
    scratch_types=[
        pltpu.VMEM((SUB_PER_TILE,), jnp.int32),
        pltpu.VMEM((SUB_CHUNKS, SUB_CHUNK), jnp.int32),
        [pltpu.VMEM((SUB_CHUNK, EMB), jnp.float32) for _ in range(SUB_GRP)],
        [pltpu.VMEM((SUB_CHUNK, EMB), jnp.float32) for _ in range(SUB_GRP)],
        pltpu.VMEM((SUB_CHUNK, 8), jnp.float32),
        pltpu.SemaphoreType.DMA,
        pltpu.SemaphoreType.DMA,
        pltpu.SemaphoreType.DMA,
        pltpu.SemaphoreType.DMA,
        pltpu.VMEM_SHARED((GP, EMB), jnp.float32),
        pltpu.VMEM_SHARED((GP, 8), jnp.float32),
        pltpu.VMEM_SHARED((NP, EMB), jnp.float32),
    ],
)
def _pool_kernel(sub_hbm, batch_hbm, h_hbm, zp_hbm, zc_hbm, ones_hbm,
                 pool_hbm, cnt_hbm,
                 sub_all, bat_all, bufs_a, bufs_b, ones_v,
                 sga, sgb, ssa, ssb, pool_sh, cnt_sh, h_sh):
    c = lax.axis_index("c")
    s = lax.axis_index("s")
    wid = c * 16 + s
    g0 = s * GROWS_PER_TILE
    r0s = s * NROWS_PER_TILE
    pltpu.sync_copy(zp_hbm, pool_sh.at[pl.ds(g0, GROWS_PER_TILE)])
    pltpu.sync_copy(zc_hbm, cnt_sh.at[pl.ds(g0, GROWS_PER_TILE)])
    pltpu.sync_copy(h_hbm.at[pl.ds(r0s, NROWS_PER_TILE)],
                    h_sh.at[pl.ds(r0s, NROWS_PER_TILE)])
    pltpu.sync_copy(sub_hbm.at[wid], sub_all)
    pltpu.sync_copy(batch_hbm.at[pl.ds(wid * SUB_CHUNKS, SUB_CHUNKS)], bat_all)
    pltpu.sync_copy(ones_hbm, ones_v)

    def fire_gather(chunk, buf, sem):
        idx = sub_all.at[pl.ds(chunk * SUB_CHUNK, SUB_CHUNK)]
        return pltpu.async_copy(h_sh.at[idx], buf, sem)

    def fire_scatter(chunk, buf, sem):
        d1 = pltpu.async_copy(buf, pool_sh.at[bat_all.at[chunk]], sem, add=True)
        d2 = pltpu.async_copy(ones_v, cnt_sh.at[bat_all.at[chunk]], sem, add=True)
        return (d1, d2)

    plsc.subcore_barrier()

    def body(gp, _):
        base_a = gp * 2 * SUB_GRP
        base_b = base_a + SUB_GRP
        ga = [fire_gather(base_a + k, bufs_a[k], sga) for k in range(SUB_GRP)]
        gb = [fire_gather(base_b + k, bufs_b[k], sgb) for k in range(SUB_GRP)]
        for d in ga:
            d.wait()
        sa = [fire_scatter(base_a + k, bufs_a[k], ssa) for k in range(SUB_GRP)]
        for d in gb:
            d.wait()
        sb = [fire_scatter(base_b + k, bufs_b[k], ssb) for k in range(SUB_GRP)]
        for d1, d2 in sa:
            d1.wait()
            d2.wait()
        for d1, d2 in sb:
            d1.wait()
            d2.wait()
        return 0

    lax.fori_loop(0, SUB_PAIRS, body, 0)
    tg = fire_gather(SUB_CHUNKS - 1, bufs_a[0], sga)
    tg.wait()
    t1, t2 = fire_scatter(SUB_CHUNKS - 1, bufs_a[0], ssa)
    t1.wait()
    t2.wait()
    plsc.subcore_barrier()
    pltpu.sync_copy(pool_sh.at[pl.ds(g0, GROWS_PER_TILE)],
                    pool_hbm.at[c, pl.ds(g0, GROWS_PER_TILE)])
    pltpu.sync_copy(cnt_sh.at[pl.ds(g0, GROWS_PER_TILE)],
                    cnt_hbm.at[c, pl.ds(g0, GROWS_PER_TILE)])


def _head_body(pool_ref, cnt_ref, w1_ref, b1_ref, g_ref, be_ref, w2_ref, b2_ref,
               out_ref):
    pp = pool_ref[...]
    cc = cnt_ref[...]
    nreal = N_GRAPHS * BIN_REP
    pool = (pp[0, :nreal, :] + pp[1, :nreal, :]).reshape(
        N_GRAPHS, BIN_REP, EMB).sum(axis=1)
    cnt = (cc[0, :nreal, :] + cc[1, :nreal, :]).reshape(
        N_GRAPHS, BIN_REP, 8).sum(axis=1)[:, 0:1]
    pooled = pool / jnp.maximum(cnt, 1.0)
    z = jnp.dot(pooled, w1_ref[...], preferred_element_type=jnp.float32) + b1_ref[...]
    z = jnp.maximum(z, 0.0)
    mu = jnp.mean(z, axis=0, keepdims=True)
    var = jnp.mean((z - mu) ** 2, axis=0, keepdims=True)
    z = (z - mu) / jnp.sqrt(var + 1e-5) * g_ref[...] + be_ref[...]
    out_ref[...] = jnp.sum(z * w2_ref[...], axis=1, keepdims=True) + b2_ref[...]


def _head_call(pool_parts, cnt_parts, w1, b1_2d, gamma2d, beta2d, w2row, b2_2d):
    return pl.pallas_call(
        _head_body,
        out_shape=jax.ShapeDtypeStruct((N_GRAPHS, 1), jnp.float32),
    )(pool_parts, cnt_parts, w1, b1_2d, gamma2d, beta2d, w2row, b2_2d)


def kernel(full_x, full_edge_index, subgraph_node_indices, batch_vector,
           W_enc, b_enc, W1, b1, gamma, beta, W2, b2):
    src = full_edge_index[0]
    dst = full_edge_index[1]
    e_pad = EP - N_EDGES
    src_pad = jnp.concatenate([src, jnp.zeros((e_pad,), jnp.int32)])
    dst_fill = N_NODES + (jnp.arange(e_pad, dtype=jnp.int32) % (NP - N_NODES))
    dst_pad = jnp.concatenate([dst, dst_fill])
    src_2d = src_pad.reshape(N_TILES, EDGES_PER_TILE)
    dst_2d = dst_pad.reshape(N_TILES * EDGE_CHUNKS, EDGE_CHUNK)
    s_pad = SP - S_SUB
    sub_pad = jnp.concatenate([subgraph_node_indices, jnp.zeros((s_pad,), jnp.int32)])
    bins_real = batch_vector * BIN_REP + (jnp.arange(S_SUB, dtype=jnp.int32) % BIN_REP)
    bins_pad = N_GRAPHS * BIN_REP + (jnp.arange(s_pad, dtype=jnp.int32) % (GP - N_GRAPHS * BIN_REP))
    batch_pad = jnp.concatenate([bins_real, bins_pad])
    sub_2d = sub_pad.reshape(N_TILES, SUB_PER_TILE)
    batch_2d = batch_pad.reshape(N_TILES * SUB_CHUNKS, SUB_CHUNK)
    x_pad = jnp.pad(full_x, ((0, NP - N_NODES), (0, 0)))

    ones_e = jnp.ones((EDGE_CHUNK, 8), jnp.float32)
    zeros_deg = jnp.zeros((NROWS_PER_TILE, 8), jnp.float32)
    zeros_acc = jnp.zeros((NROWS_PER_TILE, EMB), jnp.float32)
    ones_s = jnp.ones((SUB_CHUNK, 8), jnp.float32)
    zeros_pool = jnp.zeros((GROWS_PER_TILE, EMB), jnp.float32)
    zeros_cnt = jnp.zeros((GROWS_PER_TILE, 8), jnp.float32)

    deg_parts = _deg_kernel(dst_2d, ones_e, zeros_deg)
    y, dinv = _enc_call(deg_parts, x_pad, W_enc)
    acc_parts = _edge_kernel(src_2d, dst_2d, y, zeros_acc)
    h = _combine_call(acc_parts, y, dinv, b_enc.reshape(1, EMB))
    pool_parts, cnt_parts = _pool_kernel(sub_2d, batch_2d, h,
                                         zeros_pool, zeros_cnt, ones_s)
    return _head_call(pool_parts, cnt_parts, W1, b1.reshape(1, HID),
                      gamma.reshape(1, HID), beta.reshape(1, HID),
                      W2.reshape(1, HID), b2.reshape(1, 1))


# speedup vs baseline: 2.3290x; 1.0453x over previous
import functools

import jax
import jax.numpy as jnp
from jax import lax
from jax.experimental import pallas as pl
from jax.experimental.pallas import tpu as pltpu
from jax.experimental.pallas import tpu_sc as plsc

N_NODES = 10000
N_EDGES = 320000
D_FEAT = 128
S_SUB = 50000
N_GRAPHS = 256
EMB = 64
HID = 64

N_TILES = 32
NROWS_PER_TILE = N_NODES // 16

EDGE_CHUNK = 128
EDGE_GRP = 2
EDGE_CHUNKS = N_EDGES // (N_TILES * EDGE_CHUNK)
EDGE_PAIRS = EDGE_CHUNKS // (2 * EDGE_GRP)
EDGE_TAIL = EDGE_CHUNKS - EDGE_PAIRS * 2 * EDGE_GRP
EDGES_PER_TILE = EDGE_CHUNKS * EDGE_CHUNK
EDGE_REM_BASE = N_TILES * EDGES_PER_TILE
EDGE_DST_ROWS = N_EDGES // EDGE_CHUNK

SUB_CHUNK = 64
SUB_GRP = 4
SUB_CHUNKS = 24
SUB_PAIRS = SUB_CHUNKS // (2 * SUB_GRP)
SUB_PER_TILE = SUB_CHUNKS * SUB_CHUNK
SUB_REM_BASE = N_TILES * SUB_PER_TILE
SUB_REM64 = 13
SUB_TAIL_BASE = SUB_REM_BASE + SUB_REM64 * SUB_CHUNK
SUB_TAIL = S_SUB - SUB_TAIL_BASE

BIN_REP = 4
GP = N_GRAPHS * BIN_REP
GROWS_PER_TILE = GP // 16

ROW_BLK = 2000


def _sc_mesh():
    return plsc.VectorSubcoreMesh(core_axis_name="c", subcore_axis_name="s")


_SC_PARAMS = pltpu.CompilerParams(use_tc_tiling_on_sc=False)


@functools.partial(
    pl.kernel,
    out_type=jax.ShapeDtypeStruct((2, N_NODES, 8), jnp.float32),
    mesh=_sc_mesh(),
    compiler_params=_SC_PARAMS,
    scratch_types=[
        pltpu.VMEM((EDGE_CHUNKS + 1, EDGE_CHUNK), jnp.int32),
        pltpu.VMEM((EDGE_CHUNK, 8), jnp.float32),
        pltpu.SemaphoreType.DMA,
        pltpu.VMEM_SHARED((N_NODES, 8), jnp.float32),
    ],
)
def _deg_kernel(dst2_hbm, ones_hbm, zeros_hbm, out_hbm, dst_all, ones_v, sem,
                deg_sh):
    c = lax.axis_index("c")
    s = lax.axis_index("s")
    wid = c * 16 + s
    r0 = s * NROWS_PER_TILE
    pltpu.sync_copy(zeros_hbm, deg_sh.at[pl.ds(r0, NROWS_PER_TILE)])
    pltpu.sync_copy(dst2_hbm.at[pl.ds(wid * EDGE_CHUNKS, EDGE_CHUNKS)],
                    dst_all.at[pl.ds(0, EDGE_CHUNKS)])

    @pl.when(wid < 4)
    def _():
        pltpu.sync_copy(dst2_hbm.at[pl.ds(N_TILES * EDGE_CHUNKS + wid, 1)],
                        dst_all.at[pl.ds(EDGE_CHUNKS, 1)])

    pltpu.sync_copy(ones_hbm, ones_v)
    plsc.subcore_barrier()

    def fire(chunk):
        return pltpu.async_copy(ones_v, deg_sh.at[dst_all.at[chunk]], sem,
                                add=True)

    def body(gp, _):
        ds = [fire(gp * 8 + k) for k in range(8)]
        for d in ds:
            d.wait()
        return 0

    lax.fori_loop(0, EDGE_CHUNKS // 8, body, 0)
    tail = [fire(72 + k) for k in range(EDGE_CHUNKS - 72)]
    for d in tail:
        d.wait()

    @pl.when(wid < 4)
    def _():
        fire(EDGE_CHUNKS).wait()

    plsc.subcore_barrier()
    pltpu.sync_copy(deg_sh.at[pl.ds(r0, NROWS_PER_TILE)],
                    out_hbm.at[c, pl.ds(r0, NROWS_PER_TILE)])


def _enc_body(dp_ref, x_ref, w_ref, y_ref, dinv_ref):
    dp = dp_ref[...]
    deg = dp[0] + dp[1] + 1.0
    dinv = lax.rsqrt(deg)
    xw = jnp.dot(x_ref[...], w_ref[...], preferred_element_type=jnp.float32)
    y_ref[...] = xw * dinv[:, 0:1]
    dinv_ref[...] = dinv


def _enc_call(deg_parts, x, w_enc):
    grid = N_NODES // ROW_BLK
    return pl.pallas_call(
        _enc_body,
        grid=(grid,),
        in_specs=[
            pl.BlockSpec((2, ROW_BLK, 8), lambda i: (0, i, 0)),
            pl.BlockSpec((ROW_BLK, D_FEAT), lambda i: (i, 0)),
            pl.BlockSpec((D_FEAT, EMB), lambda i: (0, 0)),
        ],
        out_specs=[
            pl.BlockSpec((ROW_BLK, EMB), lambda i: (i, 0)),
            pl.BlockSpec((ROW_BLK, 8), lambda i: (i, 0)),
        ],
        out_shape=[
            jax.ShapeDtypeStruct((N_NODES, EMB), jnp.float32),
            jax.ShapeDtypeStruct((N_NODES, 8), jnp.float32),
        ],
    )(deg_parts, x, w_enc)


@functools.partial(
    pl.kernel,
    out_type=jax.ShapeDtypeStruct((2, N_NODES, EMB), jnp.float32),
    mesh=_sc_mesh(),
    compiler_params=_SC_PARAMS,
    scratch_types=[
        pltpu.VMEM((EDGES_PER_TILE + EDGE_CHUNK,), jnp.int32),
        [pltpu.VMEM((EDGE_CHUNK,), jnp.int32) for _ in range(EDGE_GRP)],
        [pltpu.VMEM((EDGE_CHUNK,), jnp.int32) for _ in range(EDGE_GRP)],
        [pltpu.VMEM((EDGE_CHUNK, EMB), jnp.float32) for _ in range(EDGE_GRP)],
        [pltpu.VMEM((EDGE_CHUNK, EMB), jnp.float32) for _ in range(EDGE_GRP)],
        pltpu.SemaphoreType.DMA,
        pltpu.SemaphoreType.DMA,
        pltpu.SemaphoreType.DMA,
        pltpu.SemaphoreType.DMA,
        pltpu.SemaphoreType.DMA,
        pltpu.SemaphoreType.DMA,
        pltpu.VMEM_SHARED((N_NODES, EMB), jnp.float32),
        pltpu.VMEM_SHARED((N_NODES, EMB), jnp.float32),
    ],
)
def _edge_kernel(src_hbm, dst2_hbm, y_hbm, zeros_hbm, out_hbm,
                 src_all, dbuf_a, dbuf_b, bufs_a, bufs_b,
                 sga, sgb, ssa, ssb, sda, sdb, acc_sh, y_sh):
    c = lax.axis_index("c")
    s = lax.axis_index("s")
    wid = c * 16 + s
    r0 = s * NROWS_PER_TILE
    pltpu.sync_copy(zeros_hbm, acc_sh.at[pl.ds(r0, NROWS_PER_TILE)])
    pltpu.sync_copy(y_hbm.at[pl.ds(r0, NROWS_PER_TILE)],
                    y_sh.at[pl.ds(r0, NROWS_PER_TILE)])
    pltpu.sync_copy(src_hbm.at[pl.ds(wid * EDGES_PER_TILE, EDGES_PER_TILE)],
                    src_all.at[pl.ds(0, EDGES_PER_TILE)])

    @pl.when(wid < 4)
    def _():
        pltpu.sync_copy(src_hbm.at[pl.ds(EDGE_REM_BASE + wid * EDGE_CHUNK,
                                         EDGE_CHUNK)],
                        src_all.at[pl.ds(EDGES_PER_TILE, EDGE_CHUNK)])

    def fire_gather(chunk, buf, sem):
        idx = src_all.at[pl.ds(chunk * EDGE_CHUNK, EDGE_CHUNK)]
        return pltpu.async_copy(y_sh.at[idx], buf, sem)

    def fire_dst(row, dbuf, sem):
        return pltpu.async_copy(dst2_hbm.at[row], dbuf, sem)

    plsc.subcore_barrier()

    def run_group(chunks, rows, dbufs, bufs, sg, sd):
        dd = [fire_dst(rows[k], dbufs[k], sd) for k in range(len(chunks))]
        gg = [fire_gather(chunks[k], bufs[k], sg) for k in range(len(chunks))]
        return dd, gg

    def drain_fire_scatter(dd, gg, dbufs, bufs, ss):
        for d in gg:
            d.wait()
        for d in dd:
            d.wait()
        return [pltpu.async_copy(bufs[k], acc_sh.at[dbufs[k]], ss, add=True)
                for k in range(len(dd))]

    def body(gp, _):
        base_a = gp * 2 * EDGE_GRP
        ca = [base_a + k for k in range(EDGE_GRP)]
        cb = [base_a + EDGE_GRP + k for k in range(EDGE_GRP)]
        ra = [wid * EDGE_CHUNKS + ch for ch in ca]
        rb = [wid * EDGE_CHUNKS + ch for ch in cb]
        da, ga = run_group(ca, ra, dbuf_a, bufs_a, sga, sda)
        db, gb = run_group(cb, rb, dbuf_b, bufs_b, sgb, sdb)
        sa = drain_fire_scatter(da, ga, dbuf_a, bufs_a, ssa)
        sb = drain_fire_scatter(db, gb, dbuf_b, bufs_b, ssb)
        for d in sa:
            d.wait()
        for d in sb:
            d.wait()
        return 0

    lax.fori_loop(0, EDGE_PAIRS, body, 0)

    ct = [EDGE_PAIRS * 2 * EDGE_GRP + k for k in range(EDGE_TAIL)]
    rt = [wid * EDGE_CHUNKS + ch for ch in ct]
    dt, gt = run_group(ct, rt, dbuf_a, bufs_a, sga, sda)
    st = drain_fire_scatter(dt, gt, dbuf_a, bufs_a, ssa)
    for d in st:
        d.wait()

    @pl.when(wid < 4)
    def _():
        dr, gr = run_group([EDGE_CHUNKS], [N_TILES * EDGE_CHUNKS + wid],
                           dbuf_b, bufs_b, sgb, sdb)
        sr = drain_fire_scatter(dr, gr, dbuf_b, bufs_b, ssb)
        for d in sr:
            d.wait()

    plsc.subcore_barrier()
    pltpu.sync_copy(acc_sh.at[pl.ds(r0, NROWS_PER_TILE)],
                    out_hbm.at[c, pl.ds(r0, NROWS_PER_TILE)])


def _combine_body(a_ref, y_ref, dinv_ref, b_ref, h_ref):
    a = a_ref[...]
    acc = a[0] + a[1] + y_ref[...]
    h_ref[...] = jnp.maximum(acc * dinv_ref[...][:, 0:1] + b_ref[...], 0.0)


def _combine_call(acc_parts, y, dinv, b_enc2d):
    grid = N_NODES // ROW_BLK
    return pl.pallas_call(
        _combine_body,
        grid=(grid,),
        in_specs=[
            pl.BlockSpec((2, ROW_BLK, EMB), lambda i: (0, i, 0)),
            pl.BlockSpec((ROW_BLK, EMB), lambda i: (i, 0)),
            pl.BlockSpec((ROW_BLK, 8), lambda i: (i, 0)),
            pl.BlockSpec((1, EMB), lambda i: (0, 0)),
        ],
        out_specs=pl.BlockSpec((ROW_BLK, EMB), lambda i: (i, 0)),
        out_shape=jax.ShapeDtypeStruct((N_NODES, EMB), jnp.float32),
    )(acc_parts, y, dinv, b_enc2d)


@functools.partial(
    pl.kernel,
    out_type=[
        jax.ShapeDtypeStruct((2, GP, EMB), jnp.float32),
        jax.ShapeDtypeStruct((2, GP, 8), jnp.float32),
    ],
    mesh=_sc_mesh(),
    compiler_params=_SC_PARAMS,
    scratch_types=[
        pltpu.VMEM((SUB_PER_TILE + SUB_CHUNK,), jnp.int32),
        [pltpu.VMEM((SUB_CHUNK,), jnp.int32) for _ in range(SUB_GRP)],
        [pltpu.VMEM((SUB_CHUNK,), jnp.int32) for _ in range(SUB_GRP)],
        [pltpu.VMEM((SUB_CHUNK, EMB), jnp.float32) for _ in range(SUB_GRP)],
        [pltpu.VMEM((SUB_CHUNK, EMB), jnp.float32) for _ in range(SUB_GRP)],
        pltpu.VMEM((SUB_CHUNK, 8), jnp.float32),
        pltpu.VMEM((SUB_TAIL, EMB), jnp.float32),
        pltpu.VMEM((SUB_TAIL,), jnp.int32),
        pltpu.SemaphoreType.DMA,
        pltpu.SemaphoreType.DMA,
        pltpu.SemaphoreType.DMA,
        pltpu.SemaphoreType.DMA,
        pltpu.SemaphoreType.DMA,
        pltpu.SemaphoreType.DMA,
        pltpu.VMEM_SHARED((GP, EMB), jnp.float32),
        pltpu.VMEM_SHARED((GP, 8), jnp.float32),
        pltpu.VMEM_SHARED((N_NODES, EMB), jnp.float32),
    ],
)
def _pool_kernel(sub_hbm, bin2_hbm, h_hbm, zp_hbm, zc_hbm, ones_hbm,
                 pool_hbm, cnt_hbm,
                 sub_all, bbuf_a, bbuf_b, bufs_a, bufs_b, ones_v,
                 tbuf, tibuf, sga, sgb, ssa, ssb, sda, sdb,
                 pool_sh, cnt_sh, h_sh):
    c = lax.axis_index("c")
    s = lax.axis_index("s")
    wid = c * 16 + s
    g0 = s * GROWS_PER_TILE
    r0 = s * NROWS_PER_TILE
    pltpu.sync_copy(zp_hbm, pool_sh.at[pl.ds(g0, GROWS_PER_TILE)])
    pltpu.sync_copy(zc_hbm, cnt_sh.at[pl.ds(g0, GROWS_PER_TILE)])
    pltpu.sync_copy(h_hbm.at[pl.ds(r0, NROWS_PER_TILE)],
                    h_sh.at[pl.ds(r0, NROWS_PER_TILE)])
    pltpu.sync_copy(sub_hbm.at[pl.ds(wid * SUB_PER_TILE, SUB_PER_TILE)],
                    sub_all.at[pl.ds(0, SUB_PER_TILE)])

    @pl.when(wid < SUB_REM64)
    def _():
        pltpu.sync_copy(sub_hbm.at[pl.ds(SUB_REM_BASE + wid * SUB_CHUNK,
                                         SUB_CHUNK)],
                        sub_all.at[pl.ds(SUB_PER_TILE, SUB_CHUNK)])

    @pl.when(wid == SUB_REM64)
    def _():
        pltpu.sync_copy(sub_hbm.at[pl.ds(SUB_TAIL_BASE, SUB_TAIL)],
                        sub_all.at[pl.ds(SUB_PER_TILE, SUB_TAIL)])

    pltpu.sync_copy(ones_hbm, ones_v)

    def fire_gather(chunk, buf, sem):
        idx = sub_all.at[pl.ds(chunk * SUB_CHUNK, SUB_CHUNK)]
        return pltpu.async_copy(h_sh.at[idx], buf, sem)

    def fire_bin(row, bbuf, sem):
        return pltpu.async_copy(bin2_hbm.at[row], bbuf, sem)

    plsc.subcore_barrier()

    def run_group(chunks, rows, bbufs, bufs, sg, sd):
        bb = [fire_bin(rows[k], bbufs[k], sd) for k in range(len(chunks))]
        gg = [fire_gather(chunks[k], bufs[k], sg) for k in range(len(chunks))]
        return bb, gg

    def drain_fire_scatter(bb, gg, bbufs, bufs, ss):
        for d in gg:
            d.wait()
        for d in bb:
            d.wait()
        out = []
        for k in range(len(bb)):
            out.append(pltpu.async_copy(bufs[k], pool_sh.at[bbufs[k]], ss,
                                        add=True))
            out.append(pltpu.async_copy(ones_v, cnt_sh.at[bbufs[k]], ss,
                                        add=True))
        return out

    def body(gp, _):
        base_a = gp * 2 * SUB_GRP
        ca = [base_a + k for k in range(SUB_GRP)]
        cb = [base_a + SUB_GRP + k for k in range(SUB_GRP)]
        ra = [wid * SUB_CHUNKS + ch for ch in ca]
        rb = [wid * SUB_CHUNKS + ch for ch in cb]
        ba, ga = run_group(ca, ra, bbuf_a, bufs_a, sga, sda)
        bb, gb = run_group(cb, rb, bbuf_b, bufs_b, sgb, sdb)
        sa = drain_fire_scatter(ba, ga, bbuf_a, bufs_a, ssa)
        sb = drain_fire_scatter(bb, gb, bbuf_b, bufs_b, ssb)
        for d in sa:
            d.wait()
        for d in sb:
            d.wait()
        return 0

    lax.fori_loop(0, SUB_PAIRS, body, 0)

    @pl.when(wid < SUB_REM64)
    def _():
        br, gr = run_group([SUB_CHUNKS], [N_TILES * SUB_CHUNKS + wid],
                           bbuf_a, bufs_a, sga, sda)
        sr = drain_fire_scatter(br, gr, bbuf_a, bufs_a, ssa)
        for d in sr:
            d.wait()

    @pl.when(wid == SUB_REM64)
    def _():
        db = pltpu.async_copy(
            bin2_hbm.at[N_TILES * SUB_CHUNKS + SUB_REM64, pl.ds(0, SUB_TAIL)],
            tibuf, sda)
        gg = pltpu.async_copy(
            h_sh.at[sub_all.at[pl.ds(SUB_PER_TILE, SUB_TAIL)]], tbuf, sga)
        gg.wait()
        db.wait()
        s1 = pltpu.async_copy(tbuf, pool_sh.at[tibuf], ssa, add=True)
        s2 = pltpu.async_copy(ones_v.at[pl.ds(0, SUB_TAIL)], cnt_sh.at[tibuf],
                              ssa, add=True)
        s1.wait()
        s2.wait()

    plsc.subcore_barrier()
    pltpu.sync_copy(pool_sh.at[pl.ds(g0, GROWS_PER_TILE)],
                    pool_hbm.at[c, pl.ds(g0, GROWS_PER_TILE)])
    pltpu.sync_copy(cnt_sh.at[pl.ds(g0, GROWS_PER_TILE)],
                    cnt_hbm.at[c, pl.ds(g0, GROWS_PER_TILE)])


def _head_body(pool_ref, cnt_ref, w1_ref, b1_ref, g_ref, be_ref, w2_ref, b2_ref,
               out_ref):
    pp = pool_ref[...]
    cc = cnt_ref[...]
    pool = (pp[0] + pp[1]).reshape(N_GRAPHS, BIN_REP, EMB).sum(axis=1)
    cnt = (cc[0] + cc[1]).reshape(N_GRAPHS, BIN_REP, 8).sum(axis=1)[:, 0:1]
    pooled = pool / jnp.maximum(cnt, 1.0)
    z = jnp.dot(pooled, w1_ref[...], preferred_element_type=jnp.float32) + b1_ref[...]
    z = jnp.maximum(z, 0.0)
    mu = jnp.mean(z, axis=0, keepdims=True)
    var = jnp.mean((z - mu) ** 2, axis=0, keepdims=True)
    z = (z - mu) / jnp.sqrt(var + 1e-5) * g_ref[...] + be_ref[...]
    out_ref[...] = jnp.sum(z * w2_ref[...], axis=1, keepdims=True) + b2_ref[...]


def _head_call(pool_parts, cnt_parts, w1, b1_2d, gamma2d, beta2d, w2row, b2_2d):
    return pl.pallas_call(
        _head_body,
        out_shape=jax.ShapeDtypeStruct((N_GRAPHS, 1), jnp.float32),
    )(pool_parts, cnt_parts, w1, b1_2d, gamma2d, beta2d, w2row, b2_2d)


def kernel(full_x, full_edge_index, subgraph_node_indices, batch_vector,
           W_enc, b_enc, W1, b1, gamma, beta, W2, b2):
    src = full_edge_index[0]
    dst2d = full_edge_index[1].reshape(EDGE_DST_ROWS, EDGE_CHUNK)
    bins = batch_vector * BIN_REP + (jnp.arange(S_SUB, dtype=jnp.int32) % BIN_REP)
    nbin_rows = N_TILES * SUB_CHUNKS + SUB_REM64 + 1
    pad = nbin_rows * SUB_CHUNK - S_SUB
    bins2d = jnp.concatenate(
        [bins, jnp.zeros((pad,), jnp.int32)]).reshape(nbin_rows, SUB_CHUNK)
    sub_pad = jnp.concatenate(
        [subgraph_node_indices, jnp.zeros((pad,), jnp.int32)])

    ones_e = jnp.ones((EDGE_CHUNK, 8), jnp.float32)
    zeros_deg = jnp.zeros((NROWS_PER_TILE, 8), jnp.float32)
    zeros_acc = jnp.zeros((NROWS_PER_TILE, EMB), jnp.float32)
    ones_s = jnp.ones((SUB_CHUNK, 8), jnp.float32)
    zeros_pool = jnp.zeros((GROWS_PER_TILE, EMB), jnp.float32)
    zeros_cnt = jnp.zeros((GROWS_PER_TILE, 8), jnp.float32)

    deg_parts = _deg_kernel(dst2d, ones_e, zeros_deg)
    y, dinv = _enc_call(deg_parts, full_x, W_enc)
    acc_parts = _edge_kernel(src, dst2d, y, zeros_acc)
    h = _combine_call(acc_parts, y, dinv, b_enc.reshape(1, EMB))
    pool_parts, cnt_parts = _pool_kernel(sub_pad, bins2d, h,
                                         zeros_pool, zeros_cnt, ones_s)
    return _head_call(pool_parts, cnt_parts, W1, b1.reshape(1, HID),
                      gamma.reshape(1, HID), beta.reshape(1, HID),
                      W2.reshape(1, HID), b2.reshape(1, 1))
